# baseline (speedup 1.0000x reference)
import functools

import jax
import jax.numpy as jnp
from jax.experimental import pallas as pl
from jax.experimental.pallas import tpu as pltpu

BE = 512


def _mlp_body(hs_ref, ef_ref, hd_ref, w0_ref, b0_ref, w1_ref, b1_ref,
              w2_ref, b2_ref, out_ref):
    x = (jnp.dot(hs_ref[...], w0_ref[0:64, :], preferred_element_type=jnp.float32)
         + jnp.dot(ef_ref[...], w0_ref[64:128, :], preferred_element_type=jnp.float32)
         + jnp.dot(hd_ref[...], w0_ref[128:192, :], preferred_element_type=jnp.float32)
         + b0_ref[...])
    x = jnp.dot(x, w1_ref[...], preferred_element_type=jnp.float32) + b1_ref[...]
    x = jnp.dot(x, w2_ref[...], preferred_element_type=jnp.float32) + b2_ref[...]
    out_ref[...] = x


def _filler(h_src, edge_features, h_dst, W0, b0, W1, b1, W2, b2):
    e = h_src.shape[0]
    grid = e // BE
    return pl.pallas_call(
        _mlp_body,
        grid=(grid,),
        in_specs=[
            pl.BlockSpec((BE, 64), lambda i: (i, 0)),
            pl.BlockSpec((BE, 64), lambda i: (i, 0)),
            pl.BlockSpec((BE, 64), lambda i: (i, 0)),
            pl.BlockSpec((192, 64), lambda i: (0, 0)),
            pl.BlockSpec((64,), lambda i: (0,)),
            pl.BlockSpec((64, 64), lambda i: (0, 0)),
            pl.BlockSpec((64,), lambda i: (0,)),
            pl.BlockSpec((64, 64), lambda i: (0, 0)),
            pl.BlockSpec((64,), lambda i: (0,)),
        ],
        out_specs=pl.BlockSpec((BE, 64), lambda i: (i, 0)),
        out_shape=jax.ShapeDtypeStruct((e, 64), jnp.float32),
    )(h_src, edge_features, h_dst, W0, b0, W1, b1, W2, b2)


def kernel(node_features, edge_features, edge_index, W0, b0, W1, b1, W2, b2):
    n = node_features.shape[0]
    e = edge_features.shape[0]
    src = edge_index[0]
    dst = edge_index[1]
    ep = ((e + BE - 1) // BE) * BE
    pad = ep - e
    src_p = jnp.concatenate([src, jnp.full((pad,), n, jnp.int32)])
    dst_p = jnp.concatenate([dst, jnp.zeros((pad,), jnp.int32)])
    h_src = jnp.take(node_features, jnp.minimum(src_p, n - 1), axis=0)
    h_dst = jnp.take(node_features, dst_p, axis=0)
    ef_p = jnp.concatenate([edge_features, jnp.zeros((pad, 64), jnp.float32)])
    filler = _filler(h_src, ef_p, h_dst, W0, b0, W1, b1, W2, b2)
    outer = filler[:, :, None] * h_dst[:, None, :]
    mem = jnp.zeros((n, 64, 64), jnp.float32)
    return mem.at[src_p].add(outer, mode="drop")


# speedup vs baseline: 2.3340x; 2.3340x over previous
import functools

import jax
import jax.numpy as jnp
from jax import lax
from jax.experimental import pallas as pl
from jax.experimental.pallas import tpu as pltpu

N = 10000
E = 30000
D = 64
BN = 256
BE = 256
NB = (N + BN - 1) // BN
NEB = E // BE + NB + 3
EP = NEB * BE
PADV = 1 << 30


def _routing(src, dst):
    perm = jnp.argsort(src)
    src_s = jnp.take(src, perm)
    dst_s = jnp.take(dst, perm)
    bstart = jnp.searchsorted(src_s, jnp.arange(NB + 1, dtype=jnp.int32) * BN)
    cb = bstart[1:] - bstart[:-1]
    nblk = jnp.maximum(1, (cb + BE - 1) // BE)
    blk_start = jnp.concatenate([jnp.zeros((1,), jnp.int32),
                                 jnp.cumsum(nblk)]).astype(jnp.int32)
    g = jnp.arange(NEB, dtype=jnp.int32)
    obi = jnp.clip(jnp.searchsorted(blk_start, g, side="right") - 1, 0, NB - 1)
    j = jnp.arange(EP, dtype=jnp.int32)
    bj = obi[j // BE]
    q = j - blk_start[bj] * BE
    valid = q < cb[bj]
    e_id = jnp.clip(bstart[bj] + q, 0, E - 1)
    src_pad = jnp.where(valid, src_s[e_id], PADV)
    dst_pad = jnp.where(valid, dst_s[e_id], 0)
    gid_pad = jnp.where(valid, perm[e_id], 0)
    return src_pad, dst_pad, gid_pad, obi.astype(jnp.int32)


def _tc_body(obi_ref, hs_ref, ef_ref, hd_ref, srcl_ref, w0_ref, b0_ref,
             w1_ref, b1_ref, w2_ref, b2_ref, out_ref):
    g = pl.program_id(0)
    w01 = jnp.dot(w0_ref[...], w1_ref[...], preferred_element_type=jnp.float32)
    w = jnp.dot(w01, w2_ref[...], preferred_element_type=jnp.float32)
    bias = (jnp.dot(jnp.dot(b0_ref[...], w1_ref[...],
                            preferred_element_type=jnp.float32) + b1_ref[...],
                    w2_ref[...], preferred_element_type=jnp.float32)
            + b2_ref[...])
    x = jnp.concatenate([hs_ref[...], ef_ref[...], hd_ref[...]], axis=1)
    filler = jnp.dot(x, w, preferred_element_type=jnp.float32) + bias
    hd = hd_ref[...]
    k = (filler[:, :, None] * hd[:, None, :]).reshape(BE, D * D)
    base = obi_ref[g] * BN
    sl = srcl_ref[0, 0, :] - base
    nn = lax.broadcasted_iota(jnp.int32, (BN, BE), 0)
    s = (nn == sl[None, :]).astype(jnp.float32)
    first = jnp.logical_or(g == 0, obi_ref[g] != obi_ref[jnp.maximum(g - 1, 0)])

    @pl.when(first)
    def _():
        out_ref[...] = jnp.zeros_like(out_ref)

    out_ref[...] += jnp.dot(s, k, preferred_element_type=jnp.float32)


def _tc_scatter(obi, hs, ef, hd, src_pad, W0, b0, W1, b1, W2, b2):
    grid_spec = pltpu.PrefetchScalarGridSpec(
        num_scalar_prefetch=1,
        grid=(NEB,),
        in_specs=[
            pl.BlockSpec((BE, D), lambda g, obi: (g, 0)),
            pl.BlockSpec((BE, D), lambda g, obi: (g, 0)),
            pl.BlockSpec((BE, D), lambda g, obi: (g, 0)),
            pl.BlockSpec((1, 1, BE), lambda g, obi: (g, 0, 0)),
            pl.BlockSpec((3 * D, D), lambda g, obi: (0, 0)),
            pl.BlockSpec((1, D), lambda g, obi: (0, 0)),
            pl.BlockSpec((D, D), lambda g, obi: (0, 0)),
            pl.BlockSpec((1, D), lambda g, obi: (0, 0)),
            pl.BlockSpec((D, D), lambda g, obi: (0, 0)),
            pl.BlockSpec((1, D), lambda g, obi: (0, 0)),
        ],
        out_specs=pl.BlockSpec((BN, D * D), lambda g, obi: (obi[g], 0)),
    )
    return pl.pallas_call(
        _tc_body,
        grid_spec=grid_spec,
        out_shape=jax.ShapeDtypeStruct((N, D * D), jnp.float32),
    )(obi, hs, ef, hd, src_pad.reshape(NEB, 1, BE), W0,
      b0.reshape(1, D), W1, b1.reshape(1, D), W2, b2.reshape(1, D))


def kernel(node_features, edge_features, edge_index, W0, b0, W1, b1, W2, b2):
    src = edge_index[0].astype(jnp.int32)
    dst = edge_index[1].astype(jnp.int32)
    src_pad, dst_pad, gid_pad, obi = _routing(src, dst)
    src_c = jnp.minimum(src_pad, N - 1)
    hs = jnp.take(node_features, src_c, axis=0)
    hd = jnp.take(node_features, dst_pad, axis=0)
    efp = jnp.take(edge_features, gid_pad, axis=0)
    out = _tc_scatter(obi, hs, efp, hd, src_pad, W0, b0, W1, b1, W2, b2)
    return out.reshape(N, D, D)


# speedup vs baseline: 2.3753x; 1.0177x over previous
import functools

import jax
import jax.numpy as jnp
from jax import lax
from jax.experimental import pallas as pl
from jax.experimental.pallas import tpu as pltpu
from jax.experimental.pallas import tpu_sc as plsc

N = 10000
E = 30000
D = 64
BN = 256
BE = 256
NB = (N + BN - 1) // BN
NEB = E // BE + NB + 3
EP = NEB * BE
PADV = 1 << 30


def _routing(src, dst):
    perm = jnp.argsort(src)
    src_s = jnp.take(src, perm)
    dst_s = jnp.take(dst, perm)
    bstart = jnp.searchsorted(src_s, jnp.arange(NB + 1, dtype=jnp.int32) * BN)
    cb = bstart[1:] - bstart[:-1]
    nblk = jnp.maximum(1, (cb + BE - 1) // BE)
    blk_start = jnp.concatenate([jnp.zeros((1,), jnp.int32),
                                 jnp.cumsum(nblk)]).astype(jnp.int32)
    g = jnp.arange(NEB, dtype=jnp.int32)
    obi = jnp.clip(jnp.searchsorted(blk_start, g, side="right") - 1, 0, NB - 1)
    j = jnp.arange(EP, dtype=jnp.int32)
    bj = obi[j // BE]
    q = j - blk_start[bj] * BE
    valid = q < cb[bj]
    e_id = jnp.clip(bstart[bj] + q, 0, E - 1)
    src_pad = jnp.where(valid, src_s[e_id], PADV)
    dst_pad = jnp.where(valid, dst_s[e_id], 0)
    gid_pad = jnp.where(valid, perm[e_id], 0)
    return src_pad, dst_pad, gid_pad, obi.astype(jnp.int32)


NW = 32
SPW = EP // NW
CH = 128
NCH = SPW // CH


HALF = NCH // 2


def _sc_gather_body(nf_hbm, ef_hbm, src_hbm, dst_hbm, gid_hbm,
                    hs_out, hd_out, ef_out, idx_v, rows_v, sem):
    wid = lax.axis_index("s") * 2 + lax.axis_index("c")
    base = wid * SPW
    for tbl, idx2d, out in ((nf_hbm, src_hbm, hs_out),
                            (nf_hbm, dst_hbm, hd_out),
                            (ef_hbm, gid_hbm, ef_out)):
        pltpu.sync_copy(idx2d.at[wid], idx_v)
        for r in range(2):
            copies = [
                pltpu.async_copy(tbl.at[idx_v.at[r * HALF + j]],
                                 rows_v.at[pl.ds(j * CH, CH)], sem)
                for j in range(HALF)
            ]
            for cp in copies:
                cp.wait()
            pltpu.sync_copy(rows_v,
                            out.at[pl.ds(base + r * HALF * CH, HALF * CH)])


def _sc_gather(nf_p, ef_p, src_c, dst_pad, gid_pad):
    mesh = plsc.VectorSubcoreMesh(core_axis_name="c", subcore_axis_name="s")
    row = jax.ShapeDtypeStruct((EP, 128), jnp.float32)
    fn = functools.partial(
        pl.kernel,
        mesh=mesh,
        out_type=[row, row, row],
        scratch_types=[
            pltpu.VMEM((NCH, CH), jnp.int32),
            pltpu.VMEM((HALF * CH, 128), jnp.float32),
            pltpu.SemaphoreType.DMA,
        ],
    )(_sc_gather_body)
    return fn(nf_p, ef_p,
              src_c.reshape(NW, NCH, CH), dst_pad.reshape(NW, NCH, CH),
              gid_pad.reshape(NW, NCH, CH))


def _tc_body(obi_ref, hs_ref, ef_ref, hd_ref, srcl_ref, w0_ref, b0_ref,
             w1_ref, b1_ref, w2_ref, b2_ref, out_ref):
    g = pl.program_id(0)
    hi = lax.Precision.HIGHEST
    w01 = jnp.dot(w0_ref[...], w1_ref[...], precision=hi,
                  preferred_element_type=jnp.float32)
    w = jnp.dot(w01, w2_ref[...], precision=hi,
                preferred_element_type=jnp.float32)
    bias = (jnp.dot(jnp.dot(b0_ref[...], w1_ref[...], precision=hi,
                            preferred_element_type=jnp.float32) + b1_ref[...],
                    w2_ref[...], precision=hi,
                    preferred_element_type=jnp.float32)
            + b2_ref[...])
    hd = hd_ref[:, :D]
    x = jnp.concatenate([hs_ref[:, :D], ef_ref[:, :D], hd], axis=1)
    filler = jnp.dot(x, w, precision=hi,
                     preferred_element_type=jnp.float32) + bias
    li = lax.broadcasted_iota(jnp.int32, (D, D * D), 1)
    si = lax.broadcasted_iota(jnp.int32, (D, D * D), 0)
    rep = (li // D == si).astype(jnp.bfloat16)
    til = (li % D == si).astype(jnp.bfloat16)
    fb = filler.astype(jnp.bfloat16)
    hdb = hd.astype(jnp.bfloat16)
    k = (jnp.dot(fb, rep, preferred_element_type=jnp.float32)
         * jnp.dot(hdb, til, preferred_element_type=jnp.float32)
         ).astype(jnp.bfloat16)
    base = obi_ref[g] * BN
    sl = srcl_ref[0, 0, :] - base
    nn = lax.broadcasted_iota(jnp.int32, (BN, BE), 0)
    s = (nn == sl[None, :]).astype(jnp.bfloat16)
    first = jnp.logical_or(g == 0, obi_ref[g] != obi_ref[jnp.maximum(g - 1, 0)])

    @pl.when(first)
    def _():
        out_ref[...] = jnp.zeros_like(out_ref)

    out_ref[...] += jnp.dot(s, k, preferred_element_type=jnp.float32)


def _tc_scatter(obi, hs, ef, hd, src_pad, W0, b0, W1, b1, W2, b2):
    grid_spec = pltpu.PrefetchScalarGridSpec(
        num_scalar_prefetch=1,
        grid=(NEB,),
        in_specs=[
            pl.BlockSpec((BE, 128), lambda g, obi: (g, 0)),
            pl.BlockSpec((BE, 128), lambda g, obi: (g, 0)),
            pl.BlockSpec((BE, 128), lambda g, obi: (g, 0)),
            pl.BlockSpec((1, 1, BE), lambda g, obi: (g, 0, 0)),
            pl.BlockSpec((3 * D, D), lambda g, obi: (0, 0)),
            pl.BlockSpec((1, D), lambda g, obi: (0, 0)),
            pl.BlockSpec((D, D), lambda g, obi: (0, 0)),
            pl.BlockSpec((1, D), lambda g, obi: (0, 0)),
            pl.BlockSpec((D, D), lambda g, obi: (0, 0)),
            pl.BlockSpec((1, D), lambda g, obi: (0, 0)),
        ],
        out_specs=pl.BlockSpec((BN, D * D), lambda g, obi: (obi[g], 0)),
    )
    return pl.pallas_call(
        _tc_body,
        grid_spec=grid_spec,
        out_shape=jax.ShapeDtypeStruct((N, D * D), jnp.float32),
    )(obi, hs, ef, hd, src_pad.reshape(NEB, 1, BE), W0,
      b0.reshape(1, D), W1, b1.reshape(1, D), W2, b2.reshape(1, D))


def kernel(node_features, edge_features, edge_index, W0, b0, W1, b1, W2, b2):
    src = edge_index[0].astype(jnp.int32)
    dst = edge_index[1].astype(jnp.int32)
    src_pad, dst_pad, gid_pad, obi = _routing(src, dst)
    src_c = jnp.minimum(src_pad, N - 1)
    nf_p = jnp.pad(node_features, ((0, 0), (0, 128 - D)))
    ef_p = jnp.pad(edge_features, ((0, 0), (0, 128 - D)))
    hs, hd, efp = _sc_gather(nf_p, ef_p, src_c, dst_pad, gid_pad)
    out = _tc_scatter(obi, hs, efp, hd, src_pad, W0, b0, W1, b1, W2, b2)
    return out.reshape(N, D, D)


# speedup vs baseline: 2.3771x; 1.0008x over previous
import functools

import jax
import jax.numpy as jnp
from jax import lax
from jax.experimental import pallas as pl
from jax.experimental.pallas import tpu as pltpu
from jax.experimental.pallas import tpu_sc as plsc

N = 10000
E = 30000
D = 64
BN = 256
BE = 256
NB = (N + BN - 1) // BN
NEB = E // BE + NB + 3
EP = NEB * BE
PADV = 1 << 30


def _routing(src, dst):
    perm = jnp.argsort(src)
    src_s = jnp.take(src, perm)
    dst_s = jnp.take(dst, perm)
    bstart = jnp.searchsorted(src_s, jnp.arange(NB + 1, dtype=jnp.int32) * BN)
    cb = bstart[1:] - bstart[:-1]
    nblk = jnp.maximum(1, (cb + BE - 1) // BE)
    blk_start = jnp.concatenate([jnp.zeros((1,), jnp.int32),
                                 jnp.cumsum(nblk)]).astype(jnp.int32)
    g = jnp.arange(NEB, dtype=jnp.int32)
    obi = jnp.clip(jnp.searchsorted(blk_start, g, side="right") - 1, 0, NB - 1)
    j = jnp.arange(EP, dtype=jnp.int32)
    bj = obi[j // BE]
    q = j - blk_start[bj] * BE
    valid = q < cb[bj]
    e_id = jnp.clip(bstart[bj] + q, 0, E - 1)
    src_pad = jnp.where(valid, src_s[e_id], PADV)
    dst_pad = jnp.where(valid, dst_s[e_id], 0)
    gid_pad = jnp.where(valid, perm[e_id], 0)
    return src_pad, dst_pad, gid_pad, obi.astype(jnp.int32)


NW = 32
SPW = EP // NW
CH = 128
NCH = SPW // CH


HALF = NCH // 2


def _sc_gather_body(nf_hbm, ef_hbm, src_hbm, dst_hbm, gid_hbm,
                    hs_out, hd_out, ef_out, idx_v, rows_v, sem):
    wid = lax.axis_index("s") * 2 + lax.axis_index("c")
    base = wid * SPW
    for tbl, idx2d, out in ((nf_hbm, src_hbm, hs_out),
                            (nf_hbm, dst_hbm, hd_out),
                            (ef_hbm, gid_hbm, ef_out)):
        pltpu.sync_copy(idx2d.at[wid], idx_v)
        for r in range(2):
            copies = [
                pltpu.async_copy(tbl.at[idx_v.at[r * HALF + j]],
                                 rows_v.at[pl.ds(j * CH, CH)], sem)
                for j in range(HALF)
            ]
            for cp in copies:
                cp.wait()
            pltpu.sync_copy(rows_v,
                            out.at[pl.ds(base + r * HALF * CH, HALF * CH)])


def _sc_gather(nf_p, ef_p, src_c, dst_pad, gid_pad):
    mesh = plsc.VectorSubcoreMesh(core_axis_name="c", subcore_axis_name="s")
    row = jax.ShapeDtypeStruct((EP, 128), jnp.float32)
    fn = functools.partial(
        pl.kernel,
        mesh=mesh,
        out_type=[row, row, row],
        compiler_params=pltpu.CompilerParams(use_tc_tiling_on_sc=False),
        scratch_types=[
            pltpu.VMEM((NCH, CH), jnp.int32),
            pltpu.VMEM((HALF * CH, 128), jnp.float32),
            pltpu.SemaphoreType.DMA,
        ],
    )(_sc_gather_body)
    return fn(nf_p, ef_p,
              src_c.reshape(NW, NCH, CH), dst_pad.reshape(NW, NCH, CH),
              gid_pad.reshape(NW, NCH, CH))


def _tc_body(obi_ref, hs_ref, ef_ref, hd_ref, srcl_ref, w0_ref, b0_ref,
             w1_ref, b1_ref, w2_ref, b2_ref, out_ref):
    g = pl.program_id(0)
    hi = lax.Precision.HIGHEST
    w01 = jnp.dot(w0_ref[...], w1_ref[...], precision=hi,
                  preferred_element_type=jnp.float32)
    w = jnp.dot(w01, w2_ref[...], precision=hi,
                preferred_element_type=jnp.float32)
    bias = (jnp.dot(jnp.dot(b0_ref[...], w1_ref[...], precision=hi,
                            preferred_element_type=jnp.float32) + b1_ref[...],
                    w2_ref[...], precision=hi,
                    preferred_element_type=jnp.float32)
            + b2_ref[...])
    hd = hd_ref[:, :D]
    x = jnp.concatenate([hs_ref[:, :D], ef_ref[:, :D], hd], axis=1)
    filler = jnp.dot(x, w, precision=hi,
                     preferred_element_type=jnp.float32) + bias
    li = lax.broadcasted_iota(jnp.int32, (D, D * D), 1)
    si = lax.broadcasted_iota(jnp.int32, (D, D * D), 0)
    rep = (li // D == si).astype(jnp.bfloat16)
    til = (li % D == si).astype(jnp.bfloat16)
    fb = filler.astype(jnp.bfloat16)
    hdb = hd.astype(jnp.bfloat16)
    k = (jnp.dot(fb, rep, preferred_element_type=jnp.float32)
         * jnp.dot(hdb, til, preferred_element_type=jnp.float32)
         ).astype(jnp.bfloat16)
    base = obi_ref[g] * BN
    sl = srcl_ref[0, 0, :] - base
    nn = lax.broadcasted_iota(jnp.int32, (BN, BE), 0)
    s = (nn == sl[None, :]).astype(jnp.bfloat16)
    first = jnp.logical_or(g == 0, obi_ref[g] != obi_ref[jnp.maximum(g - 1, 0)])

    @pl.when(first)
    def _():
        out_ref[...] = jnp.zeros_like(out_ref)

    out_ref[...] += jnp.dot(s, k, preferred_element_type=jnp.float32)


def _tc_scatter(obi, hs, ef, hd, src_pad, W0, b0, W1, b1, W2, b2):
    grid_spec = pltpu.PrefetchScalarGridSpec(
        num_scalar_prefetch=1,
        grid=(NEB,),
        in_specs=[
            pl.BlockSpec((BE, 128), lambda g, obi: (g, 0)),
            pl.BlockSpec((BE, 128), lambda g, obi: (g, 0)),
            pl.BlockSpec((BE, 128), lambda g, obi: (g, 0)),
            pl.BlockSpec((1, 1, BE), lambda g, obi: (g, 0, 0)),
            pl.BlockSpec((3 * D, D), lambda g, obi: (0, 0)),
            pl.BlockSpec((1, D), lambda g, obi: (0, 0)),
            pl.BlockSpec((D, D), lambda g, obi: (0, 0)),
            pl.BlockSpec((1, D), lambda g, obi: (0, 0)),
            pl.BlockSpec((D, D), lambda g, obi: (0, 0)),
            pl.BlockSpec((1, D), lambda g, obi: (0, 0)),
        ],
        out_specs=pl.BlockSpec((BN, D * D), lambda g, obi: (obi[g], 0)),
    )
    return pl.pallas_call(
        _tc_body,
        grid_spec=grid_spec,
        out_shape=jax.ShapeDtypeStruct((N, D * D), jnp.float32),
    )(obi, hs, ef, hd, src_pad.reshape(NEB, 1, BE), W0,
      b0.reshape(1, D), W1, b1.reshape(1, D), W2, b2.reshape(1, D))


def kernel(node_features, edge_features, edge_index, W0, b0, W1, b1, W2, b2):
    src = edge_index[0].astype(jnp.int32)
    dst = edge_index[1].astype(jnp.int32)
    src_pad, dst_pad, gid_pad, obi = _routing(src, dst)
    src_c = jnp.minimum(src_pad, N - 1)
    nf_p = jnp.pad(node_features, ((0, 0), (0, 128 - D)))
    ef_p = jnp.pad(edge_features, ((0, 0), (0, 128 - D)))
    hs, hd, efp = _sc_gather(nf_p, ef_p, src_c, dst_pad, gid_pad)
    out = _tc_scatter(obi, hs, efp, hd, src_pad, W0, b0, W1, b1, W2, b2)
    return out.reshape(N, D, D)


# speedup vs baseline: 3.8307x; 1.6115x over previous
import functools

import jax
import jax.numpy as jnp
from jax import lax
from jax.experimental import pallas as pl
from jax.experimental.pallas import tpu as pltpu
from jax.experimental.pallas import tpu_sc as plsc

N = 10000
E = 30000
D = 64
BN = 256
BE = 256
NB = (N + BN - 1) // BN
NEB = E // BE + NB + 3
EP = NEB * BE
PADV = 1 << 30


def _routing(src, dst):
    perm = jnp.argsort(src)
    src_s = jnp.take(src, perm)
    dst_s = jnp.take(dst, perm)
    bstart = jnp.searchsorted(src_s, jnp.arange(NB + 1, dtype=jnp.int32) * BN)
    cb = bstart[1:] - bstart[:-1]
    nblk = jnp.maximum(1, (cb + BE - 1) // BE)
    blk_start = jnp.concatenate([jnp.zeros((1,), jnp.int32),
                                 jnp.cumsum(nblk)]).astype(jnp.int32)
    g = jnp.arange(NEB, dtype=jnp.int32)
    obi = jnp.clip(jnp.searchsorted(blk_start, g, side="right") - 1, 0, NB - 1)
    estart_g = bstart[obi] + (g - blk_start[obi]) * BE
    eend_g = bstart[obi + 1]
    e_lin = (jnp.repeat(estart_g, BE)
             + jnp.tile(jnp.arange(BE, dtype=jnp.int32), NEB))
    valid = e_lin < jnp.repeat(eend_g, BE)
    e_id = jnp.clip(e_lin, 0, E - 1)
    src_pad = jnp.where(valid, src_s[e_id], PADV)
    dst_pad = jnp.where(valid, dst_s[e_id], 0)
    gid_pad = jnp.where(valid, perm[e_id], 0)
    return src_pad, dst_pad, gid_pad, obi.astype(jnp.int32)


NW = 32
SPW = EP // NW
CH = 128
NCH = SPW // CH


def _sc_gather_body(nf_hbm, ef_hbm, src_hbm, dst_hbm, gid_hbm,
                    hs_out, hd_out, ef_out, idx_v, rows_v, sem):
    wid = lax.axis_index("s") * 2 + lax.axis_index("c")
    base = wid * SPW
    for tbl, idx2d, out in ((nf_hbm, src_hbm, hs_out),
                            (nf_hbm, dst_hbm, hd_out),
                            (ef_hbm, gid_hbm, ef_out)):
        pltpu.sync_copy(idx2d.at[wid], idx_v)
        copies = [
            pltpu.async_copy(tbl.at[idx_v.at[c]],
                             rows_v.at[pl.ds(c * CH, CH)], sem)
            for c in range(NCH)
        ]
        for cp in copies:
            cp.wait()
        pltpu.sync_copy(rows_v, out.at[pl.ds(base, SPW)])


def _sc_gather(nf_p, ef_p, src_c, dst_pad, gid_pad):
    mesh = plsc.VectorSubcoreMesh(core_axis_name="c", subcore_axis_name="s")
    row = jax.ShapeDtypeStruct((EP, D), jnp.float32)
    fn = functools.partial(
        pl.kernel,
        mesh=mesh,
        out_type=[row, row, row],
        compiler_params=pltpu.CompilerParams(use_tc_tiling_on_sc=False),
        scratch_types=[
            pltpu.VMEM((NCH, CH), jnp.int32),
            pltpu.VMEM((SPW, D), jnp.float32),
            pltpu.SemaphoreType.DMA,
        ],
    )(_sc_gather_body)
    return fn(nf_p, ef_p,
              src_c.reshape(NW, NCH, CH), dst_pad.reshape(NW, NCH, CH),
              gid_pad.reshape(NW, NCH, CH))


def _tc_body(obi_ref, hs_ref, ef_ref, hd_ref, srcl_ref, w0_ref, b0_ref,
             w1_ref, b1_ref, w2_ref, b2_ref, out_ref):
    g = pl.program_id(0)
    hi = lax.Precision.HIGHEST
    w01 = jnp.dot(w0_ref[...], w1_ref[...], precision=hi,
                  preferred_element_type=jnp.float32)
    w = jnp.dot(w01, w2_ref[...], precision=hi,
                preferred_element_type=jnp.float32)
    bias = (jnp.dot(jnp.dot(b0_ref[...], w1_ref[...], precision=hi,
                            preferred_element_type=jnp.float32) + b1_ref[...],
                    w2_ref[...], precision=hi,
                    preferred_element_type=jnp.float32)
            + b2_ref[...])
    hd = hd_ref[...]
    x = jnp.concatenate([hs_ref[...], ef_ref[...], hd], axis=1)
    filler = jnp.dot(x, w, precision=hi,
                     preferred_element_type=jnp.float32) + bias
    li = lax.broadcasted_iota(jnp.int32, (D, D * D), 1)
    si = lax.broadcasted_iota(jnp.int32, (D, D * D), 0)
    rep = (li // D == si).astype(jnp.bfloat16)
    til = (li % D == si).astype(jnp.bfloat16)
    fb = filler.astype(jnp.bfloat16)
    hdb = hd.astype(jnp.bfloat16)
    k = (jnp.dot(fb, rep, preferred_element_type=jnp.float32)
         * jnp.dot(hdb, til, preferred_element_type=jnp.float32)
         ).astype(jnp.bfloat16)
    base = obi_ref[g] * BN
    sl = srcl_ref[0, 0, :] - base
    nn = lax.broadcasted_iota(jnp.int32, (BN, BE), 0)
    s = (nn == sl[None, :]).astype(jnp.bfloat16)
    first = jnp.logical_or(g == 0, obi_ref[g] != obi_ref[jnp.maximum(g - 1, 0)])

    @pl.when(first)
    def _():
        out_ref[...] = jnp.zeros_like(out_ref)

    out_ref[...] += jnp.dot(s, k, preferred_element_type=jnp.float32)


def _tc_scatter(obi, hs, ef, hd, src_pad, W0, b0, W1, b1, W2, b2):
    grid_spec = pltpu.PrefetchScalarGridSpec(
        num_scalar_prefetch=1,
        grid=(NEB,),
        in_specs=[
            pl.BlockSpec((BE, D), lambda g, obi: (g, 0)),
            pl.BlockSpec((BE, D), lambda g, obi: (g, 0)),
            pl.BlockSpec((BE, D), lambda g, obi: (g, 0)),
            pl.BlockSpec((1, 1, BE), lambda g, obi: (g, 0, 0)),
            pl.BlockSpec((3 * D, D), lambda g, obi: (0, 0)),
            pl.BlockSpec((1, D), lambda g, obi: (0, 0)),
            pl.BlockSpec((D, D), lambda g, obi: (0, 0)),
            pl.BlockSpec((1, D), lambda g, obi: (0, 0)),
            pl.BlockSpec((D, D), lambda g, obi: (0, 0)),
            pl.BlockSpec((1, D), lambda g, obi: (0, 0)),
        ],
        out_specs=pl.BlockSpec((BN, D * D), lambda g, obi: (obi[g], 0)),
    )
    return pl.pallas_call(
        _tc_body,
        grid_spec=grid_spec,
        out_shape=jax.ShapeDtypeStruct((N, D * D), jnp.float32),
    )(obi, hs, ef, hd, src_pad.reshape(NEB, 1, BE), W0,
      b0.reshape(1, D), W1, b1.reshape(1, D), W2, b2.reshape(1, D))


def kernel(node_features, edge_features, edge_index, W0, b0, W1, b1, W2, b2):
    src = edge_index[0].astype(jnp.int32)
    dst = edge_index[1].astype(jnp.int32)
    src_pad, dst_pad, gid_pad, obi = _routing(src, dst)
    src_c = jnp.minimum(src_pad, N - 1)
    hs, hd, efp = _sc_gather(node_features, edge_features,
                             src_c, dst_pad, gid_pad)
    out = _tc_scatter(obi, hs, efp, hd, src_pad, W0, b0, W1, b1, W2, b2)
    return out.reshape(N, D, D)


# speedup vs baseline: 4.2768x; 1.1165x over previous
import functools

import jax
import jax.numpy as jnp
from jax import lax
from jax.experimental import pallas as pl
from jax.experimental.pallas import tpu as pltpu
from jax.experimental.pallas import tpu_sc as plsc

N = 10000
E = 30000
D = 64
BN = 256
BE = 256
NB = (N + BN - 1) // BN
NEB = E // BE + NB + 3
EP = NEB * BE
PADV = 1 << 30


def _routing(src, dst):
    perm = jnp.argsort(src)
    src_s = jnp.take(src, perm)
    dst_s = jnp.take(dst, perm)
    bstart = jnp.searchsorted(src_s, jnp.arange(NB + 1, dtype=jnp.int32) * BN)
    cb = bstart[1:] - bstart[:-1]
    nblk = jnp.maximum(1, (cb + BE - 1) // BE)
    blk_start = jnp.concatenate([jnp.zeros((1,), jnp.int32),
                                 jnp.cumsum(nblk)]).astype(jnp.int32)
    g = jnp.arange(NEB, dtype=jnp.int32)
    obi = jnp.clip(jnp.searchsorted(blk_start, g, side="right") - 1, 0, NB - 1)
    estart_g = bstart[obi] + (g - blk_start[obi]) * BE
    eend_g = bstart[obi + 1]
    e_lin = (jnp.repeat(estart_g, BE)
             + jnp.tile(jnp.arange(BE, dtype=jnp.int32), NEB))
    valid = e_lin < jnp.repeat(eend_g, BE)
    e_id = jnp.clip(e_lin, 0, E - 1)
    src_pad = jnp.where(valid, src_s[e_id], PADV)
    dst_pad = jnp.where(valid, dst_s[e_id], 0)
    gid_pad = jnp.where(valid, perm[e_id], 0)
    return src_pad, dst_pad, gid_pad, obi.astype(jnp.int32)


NW = 32
SPW = EP // NW
CH = 128
NCH = SPW // CH


def _sc_gather_body(nf_hbm, ef_hbm, dst_hbm, gid_hbm,
                    hd_out, ef_out, idx_v, rows_v, sem):
    wid = lax.axis_index("s") * 2 + lax.axis_index("c")
    base = wid * SPW
    for tbl, idx2d, out in ((nf_hbm, dst_hbm, hd_out),
                            (ef_hbm, gid_hbm, ef_out)):
        pltpu.sync_copy(idx2d.at[wid], idx_v)
        copies = [
            pltpu.async_copy(tbl.at[idx_v.at[c]],
                             rows_v.at[pl.ds(c * CH, CH)], sem)
            for c in range(NCH)
        ]
        for cp in copies:
            cp.wait()
        pltpu.sync_copy(rows_v, out.at[pl.ds(base, SPW)])


def _sc_gather(nf_p, ef_p, dst_pad, gid_pad):
    mesh = plsc.VectorSubcoreMesh(core_axis_name="c", subcore_axis_name="s")
    row = jax.ShapeDtypeStruct((EP, D), jnp.float32)
    fn = functools.partial(
        pl.kernel,
        mesh=mesh,
        out_type=[row, row],
        compiler_params=pltpu.CompilerParams(use_tc_tiling_on_sc=False),
        scratch_types=[
            pltpu.VMEM((NCH, CH), jnp.int32),
            pltpu.VMEM((SPW, D), jnp.float32),
            pltpu.SemaphoreType.DMA,
        ],
    )(_sc_gather_body)
    return fn(nf_p, ef_p,
              dst_pad.reshape(NW, NCH, CH), gid_pad.reshape(NW, NCH, CH))


NBP = NB * BN


def _pa_body(nf_ref, w0_ref, w1_ref, w2_ref, out_ref):
    g = pl.program_id(0)
    hi = lax.Precision.HIGHEST
    w01 = jnp.dot(w0_ref[0:D, :], w1_ref[...], precision=hi,
                  preferred_element_type=jnp.float32)
    wa = jnp.dot(w01, w2_ref[...], precision=hi,
                 preferred_element_type=jnp.float32)
    pa = jnp.dot(nf_ref[...], wa, precision=hi,
                 preferred_element_type=jnp.float32)
    rid = g * BN + lax.broadcasted_iota(jnp.int32, (BN, D), 0)
    out_ref[...] = jnp.where(rid < N, pa, 0.0)


def _pa(node_features, W0, W1, W2):
    return pl.pallas_call(
        _pa_body,
        grid=(NB,),
        in_specs=[
            pl.BlockSpec((BN, D), lambda g: (g, 0)),
            pl.BlockSpec((3 * D, D), lambda g: (0, 0)),
            pl.BlockSpec((D, D), lambda g: (0, 0)),
            pl.BlockSpec((D, D), lambda g: (0, 0)),
        ],
        out_specs=pl.BlockSpec((BN, D), lambda g: (g, 0)),
        out_shape=jax.ShapeDtypeStruct((NBP, D), jnp.float32),
    )(node_features, W0, W1, W2)


def _tc_body(obi_ref, pa_ref, ef_ref, hd_ref, srcl_ref, w0_ref, b0_ref,
             w1_ref, b1_ref, w2_ref, b2_ref, out_ref):
    g = pl.program_id(0)
    hi = lax.Precision.HIGHEST
    w01 = jnp.dot(w0_ref[...], w1_ref[...], precision=hi,
                  preferred_element_type=jnp.float32)
    w = jnp.dot(w01, w2_ref[...], precision=hi,
                preferred_element_type=jnp.float32)
    bias = (jnp.dot(jnp.dot(b0_ref[...], w1_ref[...], precision=hi,
                            preferred_element_type=jnp.float32) + b1_ref[...],
                    w2_ref[...], precision=hi,
                    preferred_element_type=jnp.float32)
            + b2_ref[...])
    base = obi_ref[g] * BN
    sl = srcl_ref[0, 0, :] - base
    nn = lax.broadcasted_iota(jnp.int32, (BN, BE), 0)
    sf = (nn == sl[None, :]).astype(jnp.float32)
    hd = hd_ref[...]
    hsw = jnp.dot(sf.T, pa_ref[...], precision=hi,
                  preferred_element_type=jnp.float32)
    filler = (hsw
              + jnp.dot(ef_ref[...], w[D:2 * D, :], precision=hi,
                        preferred_element_type=jnp.float32)
              + jnp.dot(hd, w[2 * D:, :], precision=hi,
                        preferred_element_type=jnp.float32)
              + bias)
    li = lax.broadcasted_iota(jnp.int32, (D, D * D), 1)
    si = lax.broadcasted_iota(jnp.int32, (D, D * D), 0)
    rep = (li // D == si).astype(jnp.bfloat16)
    til = (li % D == si).astype(jnp.bfloat16)
    fb = filler.astype(jnp.bfloat16)
    hdb = hd.astype(jnp.bfloat16)
    k = (jnp.dot(fb, rep, preferred_element_type=jnp.float32)
         * jnp.dot(hdb, til, preferred_element_type=jnp.float32)
         ).astype(jnp.bfloat16)
    s = sf.astype(jnp.bfloat16)
    first = jnp.logical_or(g == 0, obi_ref[g] != obi_ref[jnp.maximum(g - 1, 0)])

    @pl.when(first)
    def _():
        out_ref[...] = jnp.zeros_like(out_ref)

    out_ref[...] += jnp.dot(s, k, preferred_element_type=jnp.float32)


def _tc_scatter(obi, pa, ef, hd, src_pad, W0, b0, W1, b1, W2, b2):
    grid_spec = pltpu.PrefetchScalarGridSpec(
        num_scalar_prefetch=1,
        grid=(NEB,),
        in_specs=[
            pl.BlockSpec((BN, D), lambda g, obi: (obi[g], 0)),
            pl.BlockSpec((BE, D), lambda g, obi: (g, 0)),
            pl.BlockSpec((BE, D), lambda g, obi: (g, 0)),
            pl.BlockSpec((1, 1, BE), lambda g, obi: (g, 0, 0)),
            pl.BlockSpec((3 * D, D), lambda g, obi: (0, 0)),
            pl.BlockSpec((1, D), lambda g, obi: (0, 0)),
            pl.BlockSpec((D, D), lambda g, obi: (0, 0)),
            pl.BlockSpec((1, D), lambda g, obi: (0, 0)),
            pl.BlockSpec((D, D), lambda g, obi: (0, 0)),
            pl.BlockSpec((1, D), lambda g, obi: (0, 0)),
        ],
        out_specs=pl.BlockSpec((BN, D * D), lambda g, obi: (obi[g], 0)),
    )
    return pl.pallas_call(
        _tc_body,
        grid_spec=grid_spec,
        out_shape=jax.ShapeDtypeStruct((N, D * D), jnp.float32),
    )(obi, pa, ef, hd, src_pad.reshape(NEB, 1, BE), W0,
      b0.reshape(1, D), W1, b1.reshape(1, D), W2, b2.reshape(1, D))


def kernel(node_features, edge_features, edge_index, W0, b0, W1, b1, W2, b2):
    src = edge_index[0].astype(jnp.int32)
    dst = edge_index[1].astype(jnp.int32)
    src_pad, dst_pad, gid_pad, obi = _routing(src, dst)
    hd, efp = _sc_gather(node_features, edge_features, dst_pad, gid_pad)
    pa = _pa(node_features, W0, W1, W2)
    out = _tc_scatter(obi, pa, efp, hd, src_pad, W0, b0, W1, b1, W2, b2)
    return out.reshape(N, D, D)
